# baseline (speedup 1.0000x reference)
import functools

import jax
import jax.numpy as jnp
from jax import lax
from jax.experimental import pallas as pl
from jax.experimental.pallas import tpu as pltpu
from jax.experimental.pallas import tpu_sc as plsc

EMB = 128
VOCAB1 = 100001
UTT = 20
BATCH = 4096
NMT = 26
MPT = 64
NOUT = NMT * MPT

NC = 2
NS = 16
NW = NC * NS
PERW = BATCH // NW
NB = 16
NCH = PERW // NB
ROWS = UTT * NB
GR = 64
NG = ROWS // GR
LANES = 16


def _sc_body(utts_hbm, table_hbm, embs_hbm, idx_v, idx_c, rows0, rows1,
             out_v, sem0, sem1):
    wid = lax.axis_index("s") * NC + lax.axis_index("c")
    base = wid * PERW

    stage = [
        pltpu.async_copy(utts_hbm.at[p, pl.ds(base, PERW)],
                         idx_v.at[p], sem0)
        for p in range(UTT)
    ]
    for cp in stage:
        cp.wait()
    for ci in range(NCH):
        for p in range(UTT):
            flat = ci * ROWS + p * NB
            idx_c[flat // GR, pl.ds(flat % GR, LANES)] = (
                idx_v[p, pl.ds(ci * NB, LANES)] + (p * VOCAB1)
            )

    def fire(ci, rows_ref, sem):
        for q in range(NG):
            pltpu.async_copy(
                table_hbm.at[idx_c.at[ci * NG + q]],
                rows_ref.at[pl.ds(q * GR, GR)], sem)

    def drain(ci, rows_ref, sem):
        for q in range(NG):
            pltpu.make_async_copy(
                table_hbm.at[idx_c.at[ci * NG + q]],
                rows_ref.at[pl.ds(q * GR, GR)], sem).wait()

    def sum_chunk(ci, rows_ref):
        def bbody(bh, c2):
            for db in range(2):
                b = bh * 2 + db
                for c in range(EMB // LANES):
                    sl = pl.ds(c * LANES, LANES)
                    vs = [rows_ref[p * NB + b, sl] for p in range(UTT)]
                    while len(vs) > 1:
                        nxt = [vs[i] + vs[i + 1]
                               for i in range(0, len(vs) - 1, 2)]
                        if len(vs) % 2:
                            nxt.append(vs[-1])
                        vs = nxt
                    out_v[ci * NB + b, sl] = vs[0]
            return c2

        lax.fori_loop(0, NB // 2, bbody, 0)

    fire(0, rows0, sem0)

    def pair_body(i, carry):
        ci0 = 2 * i
        fire(ci0 + 1, rows1, sem1)
        drain(ci0, rows0, sem0)
        sum_chunk(ci0, rows0)

        @pl.when(ci0 + 2 < NCH)
        def _():
            fire(ci0 + 2, rows0, sem0)

        drain(ci0 + 1, rows1, sem1)
        sum_chunk(ci0 + 1, rows1)
        return carry

    lax.fori_loop(0, NCH // 2, pair_body, 0)
    pltpu.sync_copy(out_v, embs_hbm.at[pl.ds(base, PERW)])


@functools.partial(
    pl.kernel,
    mesh=plsc.VectorSubcoreMesh(core_axis_name="c", subcore_axis_name="s"),
    out_type=jax.ShapeDtypeStruct((BATCH, EMB), jnp.float32),
    scratch_types=[
        pltpu.VMEM((UTT, PERW), jnp.int32),
        pltpu.VMEM((UTT * PERW // GR, GR), jnp.int32),
        pltpu.VMEM((ROWS, EMB), jnp.float32),
        pltpu.VMEM((ROWS, EMB), jnp.float32),
        pltpu.VMEM((PERW, EMB), jnp.float32),
        pltpu.SemaphoreType.DMA,
        pltpu.SemaphoreType.DMA,
    ],
)
def _sc_gather_sum(utts_hbm, table_hbm, embs_hbm, idx_v, idx_c, rows0, rows1,
                   out_v, sem0, sem1):
    _sc_body(utts_hbm, table_hbm, embs_hbm, idx_v, idx_c, rows0, rows1,
             out_v, sem0, sem1)


def _tc_body(e_ref, w_ref, b_ref, o_ref):
    x = jnp.tanh(e_ref[...])
    y = lax.dot_general(
        w_ref[...], x, (((1,), (1,)), ((), ())),
        preferred_element_type=jnp.float32)
    o_ref[...] = y + b_ref[...]


_TB = 512


def _tc_dense_t(embs, W2t, b2col):
    return pl.pallas_call(
        _tc_body,
        grid=(BATCH // _TB,),
        in_specs=[
            pl.BlockSpec((_TB, EMB), lambda i: (i, 0)),
            pl.BlockSpec((NOUT, EMB), lambda i: (0, 0)),
            pl.BlockSpec((NOUT, 1), lambda i: (0, 0)),
        ],
        out_specs=pl.BlockSpec((NOUT, _TB), lambda i: (0, i)),
        out_shape=jax.ShapeDtypeStruct((NOUT, BATCH), jnp.float32),
    )(embs, W2t, b2col)


def kernel(utts, emb_table, W2, b2):
    embs = _sc_gather_sum(utts, emb_table)
    yt = _tc_dense_t(embs, W2.T, b2.reshape(NOUT, 1))
    return yt.reshape(NMT, MPT, BATCH).transpose(2, 0, 1)


# speedup vs baseline: 1.0495x; 1.0495x over previous
import functools

import jax
import jax.numpy as jnp
from jax import lax
from jax.experimental import pallas as pl
from jax.experimental.pallas import tpu as pltpu
from jax.experimental.pallas import tpu_sc as plsc

EMB = 128
VOCAB1 = 100001
UTT = 20
BATCH = 4096
NMT = 26
MPT = 64
NOUT = NMT * MPT

NC = 2
NS = 16
NW = NC * NS
PERW = BATCH // NW
NB = 16
NCH = PERW // NB
ROWS = UTT * NB
GR = 64
NG = ROWS // GR
LANES = 16


def _sc_body(utts_hbm, table_hbm, embs_hbm, idx_v, idx_c, rows0, rows1,
             out_v, sem0, sem1):
    wid = lax.axis_index("s") * NC + lax.axis_index("c")
    base = wid * PERW

    stage = [
        pltpu.async_copy(utts_hbm.at[p, pl.ds(base, PERW)],
                         idx_v.at[p], sem0)
        for p in range(UTT)
    ]
    for cp in stage:
        cp.wait()
    for ci in range(NCH):
        for p in range(UTT):
            flat = ci * ROWS + p * NB
            idx_c[flat // GR, pl.ds(flat % GR, LANES)] = (
                idx_v[p, pl.ds(ci * NB, LANES)] + (p * VOCAB1)
            )

    def fire(ci, rows_ref, sem):
        for q in range(NG):
            pltpu.async_copy(
                table_hbm.at[idx_c.at[ci * NG + q]],
                rows_ref.at[pl.ds(q * GR, GR)], sem)

    def drain(ci, rows_ref, sem):
        for q in range(NG):
            pltpu.make_async_copy(
                table_hbm.at[idx_c.at[ci * NG + q]],
                rows_ref.at[pl.ds(q * GR, GR)], sem).wait()

    def sum_chunk(ci, rows_ref):
        def bbody(b, c2):
            for c in range(EMB // LANES):
                sl = pl.ds(c * LANES, LANES)
                vs = [rows_ref[p * NB + b, sl] for p in range(UTT)]
                while len(vs) > 1:
                    nxt = [vs[i] + vs[i + 1] for i in range(0, len(vs) - 1, 2)]
                    if len(vs) % 2:
                        nxt.append(vs[-1])
                    vs = nxt
                out_v[ci * NB + b, sl] = vs[0]
            return c2

        lax.fori_loop(0, NB, bbody, 0)

    fire(0, rows0, sem0)

    def pair_body(i, carry):
        ci0 = 2 * i
        fire(ci0 + 1, rows1, sem1)
        drain(ci0, rows0, sem0)
        sum_chunk(ci0, rows0)

        @pl.when(ci0 + 2 < NCH)
        def _():
            fire(ci0 + 2, rows0, sem0)

        drain(ci0 + 1, rows1, sem1)
        sum_chunk(ci0 + 1, rows1)
        return carry

    lax.fori_loop(0, NCH // 2, pair_body, 0)
    pltpu.sync_copy(out_v, embs_hbm.at[pl.ds(base, PERW)])


@functools.partial(
    pl.kernel,
    mesh=plsc.VectorSubcoreMesh(core_axis_name="c", subcore_axis_name="s"),
    out_type=jax.ShapeDtypeStruct((BATCH, EMB), jnp.float32),
    scratch_types=[
        pltpu.VMEM((UTT, PERW), jnp.int32),
        pltpu.VMEM((UTT * PERW // GR, GR), jnp.int32),
        pltpu.VMEM((ROWS, EMB), jnp.float32),
        pltpu.VMEM((ROWS, EMB), jnp.float32),
        pltpu.VMEM((PERW, EMB), jnp.float32),
        pltpu.SemaphoreType.DMA,
        pltpu.SemaphoreType.DMA,
    ],
)
def _sc_gather_sum(utts_hbm, table_hbm, embs_hbm, idx_v, idx_c, rows0, rows1,
                   out_v, sem0, sem1):
    _sc_body(utts_hbm, table_hbm, embs_hbm, idx_v, idx_c, rows0, rows1,
             out_v, sem0, sem1)


def _tc_body(e_ref, w_ref, b_ref, o_ref):
    x = jnp.tanh(e_ref[...])
    y = lax.dot_general(
        w_ref[...], x, (((1,), (1,)), ((), ())),
        preferred_element_type=jnp.float32)
    o_ref[...] = y + b_ref[...]


_TB = 1024


def _tc_dense_t(embs, W2t, b2col):
    return pl.pallas_call(
        _tc_body,
        grid=(BATCH // _TB,),
        in_specs=[
            pl.BlockSpec((_TB, EMB), lambda i: (i, 0)),
            pl.BlockSpec((NOUT, EMB), lambda i: (0, 0)),
            pl.BlockSpec((NOUT, 1), lambda i: (0, 0)),
        ],
        out_specs=pl.BlockSpec((NOUT, _TB), lambda i: (0, i)),
        out_shape=jax.ShapeDtypeStruct((NOUT, BATCH), jnp.float32),
    )(embs, W2t, b2col)


def kernel(utts, emb_table, W2, b2):
    embs = _sc_gather_sum(utts, emb_table)
    yt = _tc_dense_t(embs, W2.T, b2.reshape(NOUT, 1))
    return yt.reshape(NMT, MPT, BATCH).transpose(2, 0, 1)
